# baseline (speedup 1.0000x reference)
import functools

import jax
import jax.numpy as jnp
from jax import lax
from jax.experimental import pallas as pl
from jax.experimental.pallas import tpu as pltpu
from jax.experimental.pallas import tpu_sc as plsc

B, S, C, T = 16, 4, 2, 131072
R = B * S * C
P = B * S
NW = 32
RPW = R // NW
CHB = 65536
NCH = T // CHB


def _flip_body(y_hbm, left_hbm, out_hbm, left_v, sem):
    cid = lax.axis_index("c")
    sid = lax.axis_index("s")
    w = sid * 2 + cid
    base = w * RPW

    pltpu.sync_copy(left_hbm, left_v.at[pl.ds(0, P)])
    pair0 = base // 2
    lv = left_v[pl.ds(pair0, 16)]
    l0 = lv[0]
    l1 = lv[1]

    srcs = [base + l0, base + 1 - l0, base + 2 + l1, base + 3 - l1]
    descs = [
        pltpu.async_copy(y_hbm.at[srcs[k]], out_hbm.at[base + k], sem)
        for k in range(RPW)
    ]
    for d in descs:
        d.wait()


@jax.jit
def _flip(y2, lf):
    mesh = plsc.VectorSubcoreMesh(core_axis_name="c", subcore_axis_name="s")
    return pl.kernel(
        _flip_body,
        out_type=jax.ShapeDtypeStruct((R, T), jnp.float32),
        mesh=mesh,
        scratch_types=[
            pltpu.VMEM((P + 16,), jnp.int32),
            pltpu.SemaphoreType.DMA,
        ],
    )(y2, lf)


def kernel(y, left):
    y2 = y.reshape(R, T)
    lf = left.reshape(P).astype(jnp.int32)
    out = _flip(y2, lf)
    return out.reshape(B, S, C, T)


# speedup vs baseline: 7.9497x; 7.9497x over previous
import functools

import jax
import jax.numpy as jnp
from jax import lax
from jax.experimental import pallas as pl
from jax.experimental.pallas import tpu as pltpu
from jax.experimental.pallas import tpu_sc as plsc

B, S, C, T = 16, 4, 2, 131072
R = B * S * C
P = B * S
NW = 32
RPW = R // NW
CHB = 32768
NCH = T // CHB
NB = 3


def _flip_body(y_hbm, left_hbm, out_hbm, left_v,
               buf0, buf1, buf2, rs0, rs1, rs2, ws0, ws1, ws2):
    bufs = [buf0, buf1, buf2]
    rsems = [rs0, rs1, rs2]
    wsems = [ws0, ws1, ws2]

    cid = lax.axis_index("c")
    sid = lax.axis_index("s")
    w = sid * 2 + cid
    base = w * RPW

    pltpu.sync_copy(left_hbm, left_v.at[pl.ds(0, P)])
    pair0 = base // 2
    lv = left_v[pl.ds(pair0, 16)]
    l0 = lv[0]
    l1 = lv[1]

    srcs = [base + l0, base + 1 - l0, base + 2 + l1, base + 3 - l1]
    xfers = [
        (srcs[k], base + k, j * CHB)
        for k in range(RPW)
        for j in range(NCH)
    ]
    n = len(xfers)

    rdesc = [None] * NB
    wdesc = [None] * NB
    for t in range(NB):
        s, _, col = xfers[t]
        rdesc[t] = pltpu.async_copy(
            y_hbm.at[s, pl.ds(col, CHB)], bufs[t], rsems[t]
        )
    for t in range(n):
        b = t % NB
        if t >= NB:
            wdesc[b].wait()
            s, _, col = xfers[t]
            rdesc[b] = pltpu.async_copy(
                y_hbm.at[s, pl.ds(col, CHB)], bufs[b], rsems[b]
            )
        rdesc[b].wait()
        _, d, col = xfers[t]
        wdesc[b] = pltpu.async_copy(
            bufs[b], out_hbm.at[d, pl.ds(col, CHB)], wsems[b]
        )
    for t in range(n - NB, n):
        wdesc[t % NB].wait()


@jax.jit
def _flip(y2, lf):
    mesh = plsc.VectorSubcoreMesh(core_axis_name="c", subcore_axis_name="s")
    return pl.kernel(
        _flip_body,
        out_type=jax.ShapeDtypeStruct((R, T), jnp.float32),
        mesh=mesh,
        scratch_types=[
            pltpu.VMEM((P + 16,), jnp.int32),
            pltpu.VMEM((CHB,), jnp.float32),
            pltpu.VMEM((CHB,), jnp.float32),
            pltpu.VMEM((CHB,), jnp.float32),
            pltpu.SemaphoreType.DMA,
            pltpu.SemaphoreType.DMA,
            pltpu.SemaphoreType.DMA,
            pltpu.SemaphoreType.DMA,
            pltpu.SemaphoreType.DMA,
            pltpu.SemaphoreType.DMA,
        ],
    )(y2, lf)


def kernel(y, left):
    y2 = y.reshape(R, T)
    lf = left.reshape(P).astype(jnp.int32)
    out = _flip(y2, lf)
    return out.reshape(B, S, C, T)


# speedup vs baseline: 33.4392x; 4.2063x over previous
import functools

import jax
import jax.numpy as jnp
from jax import lax
from jax.experimental import pallas as pl
from jax.experimental.pallas import tpu as pltpu
from jax.experimental.pallas import tpu_sc as plsc

B, S, C, T = 16, 4, 2, 131072
P = B * S
NW = 32
CHB = 32768
NCH = T // CHB
NB = 3


def _flip_body(y_hbm, left_hbm, out_hbm, left_v,
               buf0, buf1, buf2, rs0, rs1, rs2, ws0, ws1, ws2):
    bufs = [buf0, buf1, buf2]
    rsems = [rs0, rs1, rs2]
    wsems = [ws0, ws1, ws2]

    cid = lax.axis_index("c")
    sid = lax.axis_index("s")
    w = sid * 2 + cid
    pair0 = 2 * w
    b = pair0 // S
    s0 = pair0 % S
    s1 = s0 + 1

    pltpu.sync_copy(left_hbm, left_v.at[pl.ds(0, P)])
    lv = left_v[pl.ds(pair0, 16)]
    l0 = lv[0]
    l1 = lv[1]

    rows = [(s0, 0, l0), (s0, 1, 1 - l0), (s1, 0, l1), (s1, 1, 1 - l1)]
    xfers = [
        (s, dc, sc, j * CHB)
        for (s, dc, sc) in rows
        for j in range(NCH)
    ]
    n = len(xfers)

    rdesc = [None] * NB
    wdesc = [None] * NB
    for t in range(NB):
        s, _, sc, col = xfers[t]
        rdesc[t] = pltpu.async_copy(
            y_hbm.at[b, s, sc, pl.ds(col, CHB)], bufs[t], rsems[t]
        )
    for t in range(n):
        bb = t % NB
        if t >= NB:
            wdesc[bb].wait()
            s, _, sc, col = xfers[t]
            rdesc[bb] = pltpu.async_copy(
                y_hbm.at[b, s, sc, pl.ds(col, CHB)], bufs[bb], rsems[bb]
            )
        rdesc[bb].wait()
        s, dc, _, col = xfers[t]
        wdesc[bb] = pltpu.async_copy(
            bufs[bb], out_hbm.at[b, s, dc, pl.ds(col, CHB)], wsems[bb]
        )
    for t in range(n - NB, n):
        wdesc[t % NB].wait()


@jax.jit
def _flip(y, lf):
    mesh = plsc.VectorSubcoreMesh(core_axis_name="c", subcore_axis_name="s")
    return pl.kernel(
        _flip_body,
        out_type=jax.ShapeDtypeStruct((B, S, C, T), jnp.float32),
        mesh=mesh,
        scratch_types=[
            pltpu.VMEM((P + 16,), jnp.int32),
            pltpu.VMEM((CHB,), jnp.float32),
            pltpu.VMEM((CHB,), jnp.float32),
            pltpu.VMEM((CHB,), jnp.float32),
            pltpu.SemaphoreType.DMA,
            pltpu.SemaphoreType.DMA,
            pltpu.SemaphoreType.DMA,
            pltpu.SemaphoreType.DMA,
            pltpu.SemaphoreType.DMA,
            pltpu.SemaphoreType.DMA,
        ],
    )(y, lf)


def kernel(y, left):
    lf = left.reshape(P).astype(jnp.int32)
    return _flip(y, lf)
